# baseline (speedup 1.0000x reference)
import functools
import numpy as np
import jax
import jax.numpy as jnp
from jax import lax
from jax.experimental import pallas as pl
from jax.experimental.pallas import tpu as pltpu
from jax.experimental.pallas import tpu_sc as plsc

ALPHA = 0.1
THETA = 0.5

NC = 2
NS = 16
NW = NC * NS
K = 128


def _make_deg_kernel(n_pad, chunks):
    npt = n_pad // NS
    mesh = plsc.VectorSubcoreMesh(core_axis_name="c", subcore_axis_name="s")

    @functools.partial(
        pl.kernel,
        out_type=jax.ShapeDtypeStruct((NC * n_pad,), jnp.float32),
        mesh=mesh,
        scratch_types=[
            pltpu.VMEM((chunks, K), jnp.int32),
            pltpu.VMEM((K,), jnp.float32),
            pltpu.VMEM_SHARED((n_pad,), jnp.float32),
            pltpu.SemaphoreType.DMA,
        ],
    )
    def deg_kernel(col_hbm, zrow_hbm, out_hbm, col_v, ones_v, acc, sem):
        cid = lax.axis_index("c")
        sid = lax.axis_index("s")
        wid = sid * NC + cid
        pltpu.sync_copy(col_hbm.at[wid], col_v)

        def fill(i, _):
            ones_v[pl.ds(i * 16, 16)] = jnp.ones((16,), jnp.float32)
            return 0
        lax.fori_loop(0, K // 16, fill, 0)

        pltpu.sync_copy(zrow_hbm, acc.at[pl.ds(sid * npt, npt)])
        plsc.subcore_barrier()

        def step(j, _):
            pltpu.sync_copy(ones_v, acc.at[col_v.at[j]], add=True)
            return 0
        lax.fori_loop(0, chunks, step, 0)

        plsc.subcore_barrier()
        pltpu.sync_copy(acc.at[pl.ds(sid * npt, npt)],
                        out_hbm.at[pl.ds(cid * n_pad + sid * npt, npt)])

    return deg_kernel


def _make_spmm_kernel(n_pad, d, chunks):
    npt = n_pad // NS
    mesh = plsc.VectorSubcoreMesh(core_axis_name="c", subcore_axis_name="s")

    @functools.partial(
        pl.kernel,
        out_type=jax.ShapeDtypeStruct((NC, n_pad, d), jnp.float32),
        mesh=mesh,
        scratch_types=[
            pltpu.VMEM((chunks, K), jnp.int32),
            pltpu.VMEM((2, K), jnp.int32),
            pltpu.VMEM((2, K), jnp.int32),
            pltpu.VMEM((2, K, d), jnp.float32),
            pltpu.VMEM_SHARED((n_pad, d), jnp.float32),
            pltpu.SemaphoreType.DMA,
            pltpu.SemaphoreType.DMA,
        ],
    )
    def spmm_kernel(g_hbm, pk_hbm, zblk_hbm, out_hbm,
                    pk_v, rowb, colb, buf, acc, gs0, gs1):
        cid = lax.axis_index("c")
        sid = lax.axis_index("s")
        wid = sid * NC + cid
        pltpu.sync_copy(pk_hbm.at[wid], pk_v)
        pltpu.sync_copy(zblk_hbm, acc.at[pl.ds(sid * npt, npt)])
        plsc.subcore_barrier()

        gsems = (gs0, gs1)

        def unpack(j, slot):
            for i in range(K // 16):
                p = pk_v[j, pl.ds(i * 16, 16)]
                rowb[slot, pl.ds(i * 16, 16)] = lax.shift_right_logical(
                    p, 14)
                colb[slot, pl.ds(i * 16, 16)] = lax.bitwise_and(p, 16383)

        def gather(slot):
            pltpu.async_copy(g_hbm.at[rowb.at[slot]], buf.at[slot],
                             gsems[slot])

        def gwait(slot):
            pltpu.make_async_copy(g_hbm.at[rowb.at[slot]], buf.at[slot],
                                  gsems[slot]).wait()

        def scatter(slot):
            pltpu.sync_copy(buf.at[slot], acc.at[colb.at[slot]], add=True)

        unpack(0, 0)
        gather(0)

        def step(jj, _):
            j0 = jj * 2
            unpack(j0 + 1, 1)
            gather(1)
            gwait(0)
            scatter(0)

            @pl.when(j0 + 2 < chunks)
            def _():
                unpack(j0 + 2, 0)
                gather(0)
            gwait(1)
            scatter(1)
            return 0
        lax.fori_loop(0, chunks // 2, step, 0)

        plsc.subcore_barrier()
        pltpu.sync_copy(acc.at[pl.ds(sid * npt, npt)],
                        out_hbm.at[cid, pl.ds(sid * npt, npt)])

    return spmm_kernel


def _init_body(x_ref, w_ref, b_ref, d0_ref, d1_ref,
               x0_ref, g0_ref, dinv_ref):
    x0 = jnp.dot(x_ref[...], w_ref[...],
                 preferred_element_type=jnp.float32) + b_ref[...]
    deg = d0_ref[...] + d1_ref[...] + 1.0
    dinv = lax.rsqrt(deg)
    x0_ref[...] = x0
    g0_ref[...] = dinv * x0
    dinv_ref[...] = dinv


def _make_init_kernel(n_pad, d_in, d_hid, blk):
    return pl.pallas_call(
        _init_body,
        grid=(n_pad // blk,),
        in_specs=[
            pl.BlockSpec((blk, d_in), lambda i: (i, 0)),
            pl.BlockSpec((d_in, d_hid), lambda i: (0, 0)),
            pl.BlockSpec((1, d_hid), lambda i: (0, 0)),
            pl.BlockSpec((blk, 1), lambda i: (i, 0)),
            pl.BlockSpec((blk, 1), lambda i: (i, 0)),
        ],
        out_specs=[
            pl.BlockSpec((blk, d_hid), lambda i: (i, 0)),
            pl.BlockSpec((blk, d_hid), lambda i: (i, 0)),
            pl.BlockSpec((blk, 1), lambda i: (i, 0)),
        ],
        out_shape=[
            jax.ShapeDtypeStruct((n_pad, d_hid), jnp.float32),
            jax.ShapeDtypeStruct((n_pad, d_hid), jnp.float32),
            jax.ShapeDtypeStruct((n_pad, 1), jnp.float32),
        ],
    )


def _layer_body(beta, s0_ref, s1_ref, g_ref, x0_ref, dinv_ref, w_ref,
                gn_ref):
    dinv = dinv_ref[...]
    agg = dinv * (s0_ref[...] + s1_ref[...] + g_ref[...])
    out = (1.0 - ALPHA) * agg + ALPHA * x0_ref[...]
    h = (1.0 - beta) * out + beta * jnp.dot(
        out, w_ref[...], preferred_element_type=jnp.float32)
    gn_ref[...] = dinv * jnp.maximum(h, 0.0)


def _make_layer_kernel(beta, n_pad, d_hid, blk):
    return pl.pallas_call(
        functools.partial(_layer_body, beta),
        grid=(n_pad // blk,),
        in_specs=[
            pl.BlockSpec((blk, d_hid), lambda i: (i, 0)),
            pl.BlockSpec((blk, d_hid), lambda i: (i, 0)),
            pl.BlockSpec((blk, d_hid), lambda i: (i, 0)),
            pl.BlockSpec((blk, d_hid), lambda i: (i, 0)),
            pl.BlockSpec((blk, 1), lambda i: (i, 0)),
            pl.BlockSpec((d_hid, d_hid), lambda i: (0, 0)),
        ],
        out_specs=pl.BlockSpec((blk, d_hid), lambda i: (i, 0)),
        out_shape=jax.ShapeDtypeStruct((n_pad, d_hid), jnp.float32),
    )


def _final_body(beta, s0_ref, s1_ref, g_ref, x0_ref, dinv_ref, w_ref,
                w1_ref, b1_ref, y_ref):
    agg = dinv_ref[...] * (s0_ref[...] + s1_ref[...] + g_ref[...])
    out = (1.0 - ALPHA) * agg + ALPHA * x0_ref[...]
    h = (1.0 - beta) * out + beta * jnp.dot(
        out, w_ref[...], preferred_element_type=jnp.float32)
    h = jnp.maximum(h, 0.0)
    y_ref[...] = jnp.dot(h, w1_ref[...],
                         preferred_element_type=jnp.float32) + b1_ref[...]


def _make_final_kernel(beta, n_pad, d_hid, d_out, blk):
    return pl.pallas_call(
        functools.partial(_final_body, beta),
        grid=(n_pad // blk,),
        in_specs=[
            pl.BlockSpec((blk, d_hid), lambda i: (i, 0)),
            pl.BlockSpec((blk, d_hid), lambda i: (i, 0)),
            pl.BlockSpec((blk, d_hid), lambda i: (i, 0)),
            pl.BlockSpec((blk, d_hid), lambda i: (i, 0)),
            pl.BlockSpec((blk, 1), lambda i: (i, 0)),
            pl.BlockSpec((d_hid, d_hid), lambda i: (0, 0)),
            pl.BlockSpec((d_hid, d_out), lambda i: (0, 0)),
            pl.BlockSpec((1, d_out), lambda i: (0, 0)),
        ],
        out_specs=pl.BlockSpec((blk, d_out), lambda i: (i, 0)),
        out_shape=jax.ShapeDtypeStruct((n_pad, d_out), jnp.float32),
    )


@jax.jit
def kernel(x, edge_index, lin0_w, lin0_b, conv_ws, lin1_w, lin1_b):
    n, d_in = x.shape
    d_hid = lin0_w.shape[0]
    d_out = lin1_w.shape[0]
    n_l = conv_ws.shape[0]
    e = edge_index.shape[1]

    n_pad = ((n + NS * 128 - 1) // (NS * 128)) * (NS * 128)
    ept = ((e + NW * 2 * K - 1) // (NW * 2 * K)) * (2 * K)
    chunks = ept // K
    e_pad = ept * NW

    row = jnp.concatenate(
        [edge_index[0], jnp.zeros((e_pad - e,), edge_index.dtype)])
    col = jnp.concatenate(
        [edge_index[1], jnp.full((e_pad - e,), n_pad - 1, edge_index.dtype)])
    col3 = col.reshape(NW, chunks, K)
    pk3 = ((row << 14) | col).reshape(NW, chunks, K)

    x_pad = jnp.pad(x, ((0, n_pad - n), (0, 0)))
    zrow = jnp.zeros((n_pad // NS,), jnp.float32)
    zblk = jnp.zeros((n_pad // NS, d_hid), jnp.float32)

    deg = _make_deg_kernel(n_pad, chunks)(col3, zrow).reshape(NC, n_pad)
    x0, g, dinv = _make_init_kernel(n_pad, d_in, d_hid, 2048)(
        x_pad, lin0_w.T, lin0_b[None, :], deg[0][:, None], deg[1][:, None])

    spmm = _make_spmm_kernel(n_pad, d_hid, chunks)
    for i in range(n_l):
        beta = float(np.log(THETA / (i + 1) + 1.0))
        s = spmm(g, pk3, zblk)
        if i + 1 < n_l:
            g = _make_layer_kernel(beta, n_pad, d_hid, 2048)(
                s[0], s[1], g, x0, dinv, conv_ws[i])
        else:
            y = _make_final_kernel(beta, n_pad, d_hid, d_out, 2048)(
                s[0], s[1], g, x0, dinv, conv_ws[i], lin1_w.T,
                lin1_b[None, :])
    return y[:n]


# speedup vs baseline: 2.0717x; 2.0717x over previous
import functools
import numpy as np
import jax
import jax.numpy as jnp
from jax import lax
from jax.experimental import pallas as pl
from jax.experimental.pallas import tpu as pltpu
from jax.experimental.pallas import tpu_sc as plsc

ALPHA = 0.1
THETA = 0.5

NC = 2
NS = 16
NW = NC * NS
K = 128
SPLIT0 = 0.65


def _make_deg_kernel(n_pad, chunks0, chunks1):
    npt = n_pad // NS
    cmax = max(chunks0, chunks1)
    mesh = plsc.VectorSubcoreMesh(core_axis_name="c", subcore_axis_name="s")

    @functools.partial(
        pl.kernel,
        out_type=jax.ShapeDtypeStruct((NC * n_pad,), jnp.float32),
        mesh=mesh,
        scratch_types=[
            pltpu.VMEM((cmax, K), jnp.int32),
            pltpu.VMEM((K,), jnp.float32),
            pltpu.VMEM_SHARED((n_pad,), jnp.float32),
            pltpu.SemaphoreType.DMA,
        ],
    )
    def deg_kernel(col_hbm, zrow_hbm, out_hbm, col_v, ones_v, acc, sem):
        cid = lax.axis_index("c")
        sid = lax.axis_index("s")
        wid = cid * NS + sid
        nchunks = lax.select(cid == 0, chunks0, chunks1)
        pltpu.sync_copy(col_hbm.at[wid], col_v)

        def fill(i, _):
            ones_v[pl.ds(i * 16, 16)] = jnp.ones((16,), jnp.float32)
            return 0
        lax.fori_loop(0, K // 16, fill, 0)

        pltpu.sync_copy(zrow_hbm, acc.at[pl.ds(sid * npt, npt)])
        plsc.subcore_barrier()

        def step(j, _):
            pltpu.sync_copy(ones_v, acc.at[col_v.at[j]], add=True)
            return 0
        lax.fori_loop(0, nchunks, step, 0)

        plsc.subcore_barrier()
        pltpu.sync_copy(acc.at[pl.ds(sid * npt, npt)],
                        out_hbm.at[pl.ds(cid * n_pad + sid * npt, npt)])

    return deg_kernel


def _make_spmm_kernel(n_pad, d, chunks0, chunks1):
    npt = n_pad // NS
    cmax = max(chunks0, chunks1)
    mesh = plsc.VectorSubcoreMesh(core_axis_name="c", subcore_axis_name="s")

    @functools.partial(
        pl.kernel,
        out_type=jax.ShapeDtypeStruct((NC, n_pad, d), jnp.float32),
        mesh=mesh,
        scratch_types=[
            pltpu.VMEM((cmax, K), jnp.int32),
            pltpu.VMEM((cmax, K), jnp.int32),
            pltpu.VMEM((K, d), jnp.float32),
            pltpu.VMEM_SHARED((n_pad, d), jnp.float32),
            pltpu.SemaphoreType.DMA,
        ],
    )
    def spmm_kernel(g_hbm, row_hbm, col_hbm, zblk_hbm, out_hbm,
                    row_v, col_v, buf, acc, gs0):
        cid = lax.axis_index("c")
        sid = lax.axis_index("s")
        wid = cid * NS + sid
        nchunks = lax.select(cid == 0, chunks0, chunks1)
        pltpu.sync_copy(row_hbm.at[wid], row_v)
        pltpu.sync_copy(col_hbm.at[wid], col_v)
        pltpu.sync_copy(zblk_hbm, acc.at[pl.ds(sid * npt, npt)])
        plsc.subcore_barrier()

        def step(j, _):
            pltpu.async_copy(g_hbm.at[row_v.at[j]], buf, gs0).wait()
            pltpu.sync_copy(buf, acc.at[col_v.at[j]], add=True)
            return 0
        lax.fori_loop(0, nchunks, step, 0)

        plsc.subcore_barrier()
        pltpu.sync_copy(acc.at[pl.ds(sid * npt, npt)],
                        out_hbm.at[cid, pl.ds(sid * npt, npt)])

    return spmm_kernel


def _init_body(x_ref, w_ref, b_ref, d0_ref, d1_ref,
               x0_ref, g0_ref, dinv_ref):
    x0 = jnp.dot(x_ref[...], w_ref[...],
                 preferred_element_type=jnp.float32) + b_ref[...]
    deg = d0_ref[...] + d1_ref[...] + 1.0
    dinv = lax.rsqrt(deg)
    x0_ref[...] = x0
    g0_ref[...] = dinv * x0
    dinv_ref[...] = dinv


def _make_init_kernel(n_pad, d_in, d_hid, blk):
    return pl.pallas_call(
        _init_body,
        grid=(n_pad // blk,),
        in_specs=[
            pl.BlockSpec((blk, d_in), lambda i: (i, 0)),
            pl.BlockSpec((d_in, d_hid), lambda i: (0, 0)),
            pl.BlockSpec((1, d_hid), lambda i: (0, 0)),
            pl.BlockSpec((blk, 1), lambda i: (i, 0)),
            pl.BlockSpec((blk, 1), lambda i: (i, 0)),
        ],
        out_specs=[
            pl.BlockSpec((blk, d_hid), lambda i: (i, 0)),
            pl.BlockSpec((blk, d_hid), lambda i: (i, 0)),
            pl.BlockSpec((blk, 1), lambda i: (i, 0)),
        ],
        out_shape=[
            jax.ShapeDtypeStruct((n_pad, d_hid), jnp.float32),
            jax.ShapeDtypeStruct((n_pad, d_hid), jnp.float32),
            jax.ShapeDtypeStruct((n_pad, 1), jnp.float32),
        ],
    )


def _layer_body(beta, s0_ref, s1_ref, g_ref, x0_ref, dinv_ref, w_ref,
                gn_ref):
    dinv = dinv_ref[...]
    agg = dinv * (s0_ref[...] + s1_ref[...] + g_ref[...])
    out = (1.0 - ALPHA) * agg + ALPHA * x0_ref[...]
    h = (1.0 - beta) * out + beta * jnp.dot(
        out, w_ref[...], preferred_element_type=jnp.float32)
    gn_ref[...] = dinv * jnp.maximum(h, 0.0)


def _make_layer_kernel(beta, n_pad, d_hid, blk):
    return pl.pallas_call(
        functools.partial(_layer_body, beta),
        grid=(n_pad // blk,),
        in_specs=[
            pl.BlockSpec((blk, d_hid), lambda i: (i, 0)),
            pl.BlockSpec((blk, d_hid), lambda i: (i, 0)),
            pl.BlockSpec((blk, d_hid), lambda i: (i, 0)),
            pl.BlockSpec((blk, d_hid), lambda i: (i, 0)),
            pl.BlockSpec((blk, 1), lambda i: (i, 0)),
            pl.BlockSpec((d_hid, d_hid), lambda i: (0, 0)),
        ],
        out_specs=pl.BlockSpec((blk, d_hid), lambda i: (i, 0)),
        out_shape=jax.ShapeDtypeStruct((n_pad, d_hid), jnp.float32),
    )


def _final_body(beta, s0_ref, s1_ref, g_ref, x0_ref, dinv_ref, w_ref,
                w1_ref, b1_ref, y_ref):
    agg = dinv_ref[...] * (s0_ref[...] + s1_ref[...] + g_ref[...])
    out = (1.0 - ALPHA) * agg + ALPHA * x0_ref[...]
    h = (1.0 - beta) * out + beta * jnp.dot(
        out, w_ref[...], preferred_element_type=jnp.float32)
    h = jnp.maximum(h, 0.0)
    y_ref[...] = jnp.dot(h, w1_ref[...],
                         preferred_element_type=jnp.float32) + b1_ref[...]


def _make_final_kernel(beta, n_pad, d_hid, d_out, blk):
    return pl.pallas_call(
        functools.partial(_final_body, beta),
        grid=(n_pad // blk,),
        in_specs=[
            pl.BlockSpec((blk, d_hid), lambda i: (i, 0)),
            pl.BlockSpec((blk, d_hid), lambda i: (i, 0)),
            pl.BlockSpec((blk, d_hid), lambda i: (i, 0)),
            pl.BlockSpec((blk, d_hid), lambda i: (i, 0)),
            pl.BlockSpec((blk, 1), lambda i: (i, 0)),
            pl.BlockSpec((d_hid, d_hid), lambda i: (0, 0)),
            pl.BlockSpec((d_hid, d_out), lambda i: (0, 0)),
            pl.BlockSpec((1, d_out), lambda i: (0, 0)),
        ],
        out_specs=pl.BlockSpec((blk, d_out), lambda i: (i, 0)),
        out_shape=jax.ShapeDtypeStruct((n_pad, d_out), jnp.float32),
    )


@jax.jit
def kernel(x, edge_index, lin0_w, lin0_b, conv_ws, lin1_w, lin1_b):
    n, d_in = x.shape
    d_hid = lin0_w.shape[0]
    d_out = lin1_w.shape[0]
    n_l = conv_ws.shape[0]
    e = edge_index.shape[1]

    n_pad = ((n + NS * 128 - 1) // (NS * 128)) * (NS * 128)
    ctot = -(-e // (NS * K))
    c0 = int(round(ctot * SPLIT0))
    c1 = ctot - c0
    cmax = max(c0, c1)
    e_pad = NS * K * ctot

    def shard(arr, fill):
        a = jnp.concatenate(
            [arr, jnp.full((e_pad - e,), fill, edge_index.dtype)])
        pa = a[:NS * c0 * K].reshape(NS, c0, K)
        pb = a[NS * c0 * K:].reshape(NS, c1, K)
        pa = jnp.pad(pa, ((0, 0), (0, cmax - c0), (0, 0)))
        pb = jnp.pad(pb, ((0, 0), (0, cmax - c1), (0, 0)))
        return jnp.concatenate([pa, pb], axis=0)

    row3 = shard(edge_index[0], 0)
    col3 = shard(edge_index[1], n_pad - 1)

    x_pad = jnp.pad(x, ((0, n_pad - n), (0, 0)))
    zrow = jnp.zeros((n_pad // NS,), jnp.float32)
    zblk = jnp.zeros((n_pad // NS, d_hid), jnp.float32)

    deg = _make_deg_kernel(n_pad, c0, c1)(col3, zrow).reshape(NC, n_pad)
    x0, g, dinv = _make_init_kernel(n_pad, d_in, d_hid, 2048)(
        x_pad, lin0_w.T, lin0_b[None, :], deg[0][:, None], deg[1][:, None])

    spmm = _make_spmm_kernel(n_pad, d_hid, c0, c1)
    for i in range(n_l):
        beta = float(np.log(THETA / (i + 1) + 1.0))
        s = spmm(g, row3, col3, zblk)
        if i + 1 < n_l:
            g = _make_layer_kernel(beta, n_pad, d_hid, 2048)(
                s[0], s[1], g, x0, dinv, conv_ws[i])
        else:
            y = _make_final_kernel(beta, n_pad, d_hid, d_out, 2048)(
                s[0], s[1], g, x0, dinv, conv_ws[i], lin1_w.T,
                lin1_b[None, :])
    return y[:n]


# speedup vs baseline: 2.1679x; 1.0464x over previous
import functools
import numpy as np
import jax
import jax.numpy as jnp
from jax import lax
from jax.experimental import pallas as pl
from jax.experimental.pallas import tpu as pltpu
from jax.experimental.pallas import tpu_sc as plsc

ALPHA = 0.1
THETA = 0.5

NC = 2
NS = 16
NW = NC * NS
K = 128
SPLIT0 = 0.61


def _make_deg_kernel(n_pad, chunks0, chunks1):
    npt = n_pad // NS
    cmax = max(chunks0, chunks1)
    mesh = plsc.VectorSubcoreMesh(core_axis_name="c", subcore_axis_name="s")

    @functools.partial(
        pl.kernel,
        out_type=jax.ShapeDtypeStruct((NC * n_pad,), jnp.float32),
        mesh=mesh,
        scratch_types=[
            pltpu.VMEM((cmax, K), jnp.int32),
            pltpu.VMEM((K,), jnp.float32),
            pltpu.VMEM_SHARED((n_pad,), jnp.float32),
            pltpu.SemaphoreType.DMA,
        ],
    )
    def deg_kernel(col_hbm, zrow_hbm, out_hbm, col_v, ones_v, acc, sem):
        cid = lax.axis_index("c")
        sid = lax.axis_index("s")
        wid = cid * NS + sid
        nchunks = lax.select(cid == 0, chunks0, chunks1)
        pltpu.sync_copy(col_hbm.at[wid], col_v)

        def fill(i, _):
            ones_v[pl.ds(i * 16, 16)] = jnp.ones((16,), jnp.float32)
            return 0
        lax.fori_loop(0, K // 16, fill, 0)

        pltpu.sync_copy(zrow_hbm, acc.at[pl.ds(sid * npt, npt)])
        plsc.subcore_barrier()

        def step(j, _):
            pltpu.sync_copy(ones_v, acc.at[col_v.at[j]], add=True)
            return 0
        lax.fori_loop(0, nchunks, step, 0)

        plsc.subcore_barrier()
        pltpu.sync_copy(acc.at[pl.ds(sid * npt, npt)],
                        out_hbm.at[pl.ds(cid * n_pad + sid * npt, npt)])

    return deg_kernel


def _make_spmm_kernel(n_pad, d, chunks0, chunks1):
    npt = n_pad // NS
    cmax = max(chunks0, chunks1)
    mesh = plsc.VectorSubcoreMesh(core_axis_name="c", subcore_axis_name="s")

    @functools.partial(
        pl.kernel,
        out_type=jax.ShapeDtypeStruct((NC, n_pad, d), jnp.float32),
        mesh=mesh,
        scratch_types=[
            pltpu.VMEM((cmax, K), jnp.int32),
            pltpu.VMEM((cmax, K), jnp.int32),
            pltpu.VMEM((K, d), jnp.float32),
            pltpu.VMEM_SHARED((n_pad, d), jnp.float32),
            pltpu.SemaphoreType.DMA,
        ],
    )
    def spmm_kernel(g_hbm, row_hbm, col_hbm, zblk_hbm, out_hbm,
                    row_v, col_v, buf, acc, gs0):
        cid = lax.axis_index("c")
        sid = lax.axis_index("s")
        wid = cid * NS + sid
        nchunks = lax.select(cid == 0, chunks0, chunks1)
        pltpu.sync_copy(row_hbm.at[wid], row_v)
        pltpu.sync_copy(col_hbm.at[wid], col_v)
        pltpu.sync_copy(zblk_hbm, acc.at[pl.ds(sid * npt, npt)])
        plsc.subcore_barrier()

        def step(j, _):
            pltpu.async_copy(g_hbm.at[row_v.at[j]], buf, gs0).wait()
            pltpu.sync_copy(buf, acc.at[col_v.at[j]], add=True)
            return 0
        lax.fori_loop(0, nchunks, step, 0)

        plsc.subcore_barrier()
        pltpu.sync_copy(acc.at[pl.ds(sid * npt, npt)],
                        out_hbm.at[cid, pl.ds(sid * npt, npt)])

    return spmm_kernel


def _init_body(x_ref, w_ref, b_ref, d0_ref, d1_ref,
               x0_ref, g0_ref, dinv_ref):
    x0 = jnp.dot(x_ref[...], w_ref[...],
                 preferred_element_type=jnp.float32) + b_ref[...]
    deg = d0_ref[...] + d1_ref[...] + 1.0
    dinv = lax.rsqrt(deg)
    x0_ref[...] = x0
    g0_ref[...] = dinv * x0
    dinv_ref[...] = dinv


def _make_init_kernel(n_pad, d_in, d_hid, blk):
    return pl.pallas_call(
        _init_body,
        grid=(n_pad // blk,),
        in_specs=[
            pl.BlockSpec((blk, d_in), lambda i: (i, 0)),
            pl.BlockSpec((d_in, d_hid), lambda i: (0, 0)),
            pl.BlockSpec((1, d_hid), lambda i: (0, 0)),
            pl.BlockSpec((blk, 1), lambda i: (i, 0)),
            pl.BlockSpec((blk, 1), lambda i: (i, 0)),
        ],
        out_specs=[
            pl.BlockSpec((blk, d_hid), lambda i: (i, 0)),
            pl.BlockSpec((blk, d_hid), lambda i: (i, 0)),
            pl.BlockSpec((blk, 1), lambda i: (i, 0)),
        ],
        out_shape=[
            jax.ShapeDtypeStruct((n_pad, d_hid), jnp.float32),
            jax.ShapeDtypeStruct((n_pad, d_hid), jnp.float32),
            jax.ShapeDtypeStruct((n_pad, 1), jnp.float32),
        ],
    )


def _layer_body(beta, s0_ref, s1_ref, g_ref, x0_ref, dinv_ref, w_ref,
                gn_ref):
    dinv = dinv_ref[...]
    agg = dinv * (s0_ref[...] + s1_ref[...] + g_ref[...])
    out = (1.0 - ALPHA) * agg + ALPHA * x0_ref[...]
    h = (1.0 - beta) * out + beta * jnp.dot(
        out, w_ref[...], preferred_element_type=jnp.float32)
    gn_ref[...] = dinv * jnp.maximum(h, 0.0)


def _make_layer_kernel(beta, n_pad, d_hid, blk):
    return pl.pallas_call(
        functools.partial(_layer_body, beta),
        grid=(n_pad // blk,),
        in_specs=[
            pl.BlockSpec((blk, d_hid), lambda i: (i, 0)),
            pl.BlockSpec((blk, d_hid), lambda i: (i, 0)),
            pl.BlockSpec((blk, d_hid), lambda i: (i, 0)),
            pl.BlockSpec((blk, d_hid), lambda i: (i, 0)),
            pl.BlockSpec((blk, 1), lambda i: (i, 0)),
            pl.BlockSpec((d_hid, d_hid), lambda i: (0, 0)),
        ],
        out_specs=pl.BlockSpec((blk, d_hid), lambda i: (i, 0)),
        out_shape=jax.ShapeDtypeStruct((n_pad, d_hid), jnp.float32),
    )


def _final_body(beta, s0_ref, s1_ref, g_ref, x0_ref, dinv_ref, w_ref,
                w1_ref, b1_ref, y_ref):
    agg = dinv_ref[...] * (s0_ref[...] + s1_ref[...] + g_ref[...])
    out = (1.0 - ALPHA) * agg + ALPHA * x0_ref[...]
    h = (1.0 - beta) * out + beta * jnp.dot(
        out, w_ref[...], preferred_element_type=jnp.float32)
    h = jnp.maximum(h, 0.0)
    y_ref[...] = jnp.dot(h, w1_ref[...],
                         preferred_element_type=jnp.float32) + b1_ref[...]


def _make_final_kernel(beta, n_pad, d_hid, d_out, blk):
    return pl.pallas_call(
        functools.partial(_final_body, beta),
        grid=(n_pad // blk,),
        in_specs=[
            pl.BlockSpec((blk, d_hid), lambda i: (i, 0)),
            pl.BlockSpec((blk, d_hid), lambda i: (i, 0)),
            pl.BlockSpec((blk, d_hid), lambda i: (i, 0)),
            pl.BlockSpec((blk, d_hid), lambda i: (i, 0)),
            pl.BlockSpec((blk, 1), lambda i: (i, 0)),
            pl.BlockSpec((d_hid, d_hid), lambda i: (0, 0)),
            pl.BlockSpec((d_hid, d_out), lambda i: (0, 0)),
            pl.BlockSpec((1, d_out), lambda i: (0, 0)),
        ],
        out_specs=pl.BlockSpec((blk, d_out), lambda i: (i, 0)),
        out_shape=jax.ShapeDtypeStruct((n_pad, d_out), jnp.float32),
    )


@jax.jit
def kernel(x, edge_index, lin0_w, lin0_b, conv_ws, lin1_w, lin1_b):
    n, d_in = x.shape
    d_hid = lin0_w.shape[0]
    d_out = lin1_w.shape[0]
    n_l = conv_ws.shape[0]
    e = edge_index.shape[1]

    n_pad = ((n + NS * 128 - 1) // (NS * 128)) * (NS * 128)
    ctot = -(-e // (NS * K))
    c0 = int(round(ctot * SPLIT0))
    c1 = ctot - c0
    cmax = max(c0, c1)
    e_pad = NS * K * ctot

    def shard(arr, fill):
        a = jnp.concatenate(
            [arr, jnp.full((e_pad - e,), fill, edge_index.dtype)])
        pa = a[:NS * c0 * K].reshape(NS, c0, K)
        pb = a[NS * c0 * K:].reshape(NS, c1, K)
        pa = jnp.pad(pa, ((0, 0), (0, cmax - c0), (0, 0)))
        pb = jnp.pad(pb, ((0, 0), (0, cmax - c1), (0, 0)))
        return jnp.concatenate([pa, pb], axis=0)

    row3 = shard(edge_index[0], 0)
    col3 = shard(edge_index[1], n_pad - 1)

    x_pad = jnp.pad(x, ((0, n_pad - n), (0, 0)))
    zrow = jnp.zeros((n_pad // NS,), jnp.float32)
    zblk = jnp.zeros((n_pad // NS, d_hid), jnp.float32)

    deg = _make_deg_kernel(n_pad, c0, c1)(col3, zrow).reshape(NC, n_pad)
    x0, g, dinv = _make_init_kernel(n_pad, d_in, d_hid, 2048)(
        x_pad, lin0_w.T, lin0_b[None, :], deg[0][:, None], deg[1][:, None])

    spmm = _make_spmm_kernel(n_pad, d_hid, c0, c1)
    for i in range(n_l):
        beta = float(np.log(THETA / (i + 1) + 1.0))
        s = spmm(g, row3, col3, zblk)
        if i + 1 < n_l:
            g = _make_layer_kernel(beta, n_pad, d_hid, 2048)(
                s[0], s[1], g, x0, dinv, conv_ws[i])
        else:
            y = _make_final_kernel(beta, n_pad, d_hid, d_out, 2048)(
                s[0], s[1], g, x0, dinv, conv_ws[i], lin1_w.T,
                lin1_b[None, :])
    return y[:n]
